# Initial kernel scaffold; baseline (speedup 1.0000x reference)
import jax
import jax.numpy as jnp
from jax.experimental import pallas as pl


def kernel(x, max_buf, p99_99_buf, p99_9_buf, p99_buf):
    raise NotImplementedError("write your pallas kernel here")


# speedup vs baseline: 64.7725x; 64.7725x over previous
import functools

import jax
import jax.numpy as jnp
from jax import lax
from jax.experimental import pallas as pl
from jax.experimental.pallas import tpu as pltpu
from jax.experimental.pallas import tpu_sc as plsc

GAMMA = 0.99

NC = 2
NS = 16
NW = NC * NS
L = 16

SHIFT = 15
NBINS = 1 << (31 - SHIFT)
CHUNK = 8192


def _make_sc_hist(n):
    per_w = n // NW
    n_chunks = per_w // CHUNK
    n_pairs = n_chunks // 2
    assert per_w * NW == n and n_chunks * CHUNK == per_w and n_pairs * 2 == n_chunks

    mesh = plsc.VectorSubcoreMesh(core_axis_name="c", subcore_axis_name="s")

    @functools.partial(
        pl.kernel,
        mesh=mesh,
        compiler_params=pltpu.CompilerParams(needs_layout_passes=False),
        out_type=[
            jax.ShapeDtypeStruct((NW, NBINS), jnp.int32),
            jax.ShapeDtypeStruct((NW, L), jnp.int32),
        ],
        scratch_types=[
            pltpu.VMEM((NBINS,), jnp.int32),
            pltpu.VMEM((CHUNK,), jnp.int32),
            pltpu.VMEM((CHUNK,), jnp.int32),
            pltpu.VMEM((L,), jnp.int32),
            pltpu.SemaphoreType.DMA,
            pltpu.SemaphoreType.DMA,
        ],
    )
    def sc_hist(x_hbm, hist_hbm, max_hbm, hist_v, buf0, buf1, max_v, sem0, sem1):
        cid = lax.axis_index("c")
        sid = lax.axis_index("s")
        wid = sid * NC + cid
        base = wid * per_w

        zero16 = jnp.zeros((L,), jnp.int32)

        def zbody(i, carry):
            hist_v[pl.ds(i * L, L)] = zero16
            return carry
        lax.fori_loop(0, NBINS // L, zbody, 0)

        ones16 = jnp.ones((L,), jnp.int32)
        absmask = jnp.full((L,), 0x7FFFFFFF, jnp.int32)

        def process(buf, mx):
            def body(i, mx):
                bits = buf[pl.ds(i * L, L)] & absmask
                mx = jnp.maximum(mx, bits)
                idx = lax.shift_right_logical(bits, SHIFT)
                plsc.addupdate_scatter(hist_v, [idx], ones16)
                return mx
            return lax.fori_loop(0, CHUNK // L, body, mx)

        def start(c, buf, sem):
            pltpu.make_async_copy(
                x_hbm.at[pl.ds(base + c * CHUNK, CHUNK)], buf, sem).start()

        def wait(buf, sem):
            pltpu.make_async_copy(
                x_hbm.at[pl.ds(base, CHUNK)], buf, sem).wait()

        start(0, buf0, sem0)
        start(1, buf1, sem1)

        def pair(p, mx):
            wait(buf0, sem0)
            mx = process(buf0, mx)

            @pl.when(p + 1 < n_pairs)
            def _():
                start(2 * p + 2, buf0, sem0)

            wait(buf1, sem1)
            mx = process(buf1, mx)

            @pl.when(p + 1 < n_pairs)
            def _():
                start(2 * p + 3, buf1, sem1)

            return mx

        mx = lax.fori_loop(0, n_pairs, pair, jnp.zeros((L,), jnp.int32))

        max_v[...] = mx
        pltpu.sync_copy(max_v, max_hbm.at[wid])
        pltpu.sync_copy(hist_v, hist_hbm.at[wid])

    return sc_hist


def _make_tc_finalize(n):
    rows = NBINS // 128
    ks = [int(round(q * n)) - 1 for q in (0.9999, 0.999, 0.99)]
    half = 1 << (SHIFT - 1)

    def fin(hist_ref, max_ref, bufs_ref, out_ref):
        h = jnp.sum(hist_ref[...], axis=0)

        li = lax.broadcasted_iota(jnp.int32, (128, 128), 0)
        lj = lax.broadcasted_iota(jnp.int32, (128, 128), 1)
        lower = (li <= lj).astype(jnp.float32)

        def exact_dot(a_i32, m):
            acc = None
            for s in (16, 8, 0):
                d = ((lax.shift_right_logical(a_i32, s) & 255)
                     .astype(jnp.float32))
                p = lax.dot_general(d, m, (((1,), (0,)), ((), ())),
                                    preferred_element_type=jnp.float32)
                acc = p if acc is None else acc * 256.0 + p
            return acc

        rowcum = exact_dot(h, lower)
        rowsum = rowcum[:, 127:128].astype(jnp.int32)

        ri = lax.broadcasted_iota(jnp.int32, (rows, rows), 0)
        rj = lax.broadcasted_iota(jnp.int32, (rows, rows), 1)
        strict = (ri > rj).astype(jnp.float32)

        def exact_dot_l(m, a_i32):
            acc = None
            for s in (16, 8, 0):
                d = ((lax.shift_right_logical(a_i32, s) & 255)
                     .astype(jnp.float32))
                p = lax.dot_general(m, d, (((1,), (0,)), ((), ())),
                                    preferred_element_type=jnp.float32)
                acc = p if acc is None else acc * 256.0 + p
            return acc

        rowpre = exact_dot_l(strict, rowsum)
        cum = rowcum + rowpre

        def bin_of(k):
            return jnp.sum((cum <= float(k)).astype(jnp.int32))

        bins = [bin_of(k) for k in ks]

        maxbits = jnp.max(max_ref[...])

        lane = lax.broadcasted_iota(jnp.int32, (1, 128), 1)
        bits = jnp.where(lane == 0, maxbits, 0)
        for j, b in enumerate(bins):
            bits = jnp.where(lane == j + 1, b * (1 << SHIFT) + half, bits)
        vals = lax.bitcast_convert_type(bits, jnp.float32)

        bufv = jnp.zeros((1, 128), jnp.float32)
        for j in range(4):
            bufv = jnp.where(lane == j, bufs_ref[j], bufv)

        out_ref[...] = bufv * GAMMA + vals * (1.0 - GAMMA)

    return pl.pallas_call(
        fin,
        out_shape=jax.ShapeDtypeStruct((1, 128), jnp.float32),
        in_specs=[
            pl.BlockSpec(memory_space=pltpu.VMEM),
            pl.BlockSpec(memory_space=pltpu.VMEM),
            pl.BlockSpec(memory_space=pltpu.SMEM),
        ],
        out_specs=pl.BlockSpec(memory_space=pltpu.VMEM),
    )


def kernel(x, max_buf, p99_99_buf, p99_9_buf, p99_buf):
    n = x.size
    xf = lax.bitcast_convert_type(x, jnp.int32).reshape(-1)
    hists, maxes = _make_sc_hist(n)(xf)
    bufs = jnp.stack([max_buf, p99_99_buf, p99_9_buf, p99_buf]).astype(jnp.float32)
    out = _make_tc_finalize(n)(hists.reshape(NW, NBINS // 128, 128), maxes, bufs)
    return (x, out[0, 0], out[0, 1], out[0, 2], out[0, 3])


# speedup vs baseline: 125.3377x; 1.9350x over previous
import functools

import jax
import jax.numpy as jnp
from jax import lax
from jax.experimental import pallas as pl
from jax.experimental.pallas import tpu as pltpu
from jax.experimental.pallas import tpu_sc as plsc

GAMMA = 0.99

NC = 2
NS = 16
NW = NC * NS
L = 16

SHIFT = 15
NBINS = 1 << (31 - SHIFT)
CHUNK = 16384


def _make_sc_hist(n):
    per_w = n // NW
    n_chunks = per_w // CHUNK
    n_pairs = n_chunks // 2
    assert per_w * NW == n and n_chunks * CHUNK == per_w and n_pairs * 2 == n_chunks

    mesh = plsc.VectorSubcoreMesh(core_axis_name="c", subcore_axis_name="s")

    @functools.partial(
        pl.kernel,
        mesh=mesh,
        compiler_params=pltpu.CompilerParams(needs_layout_passes=False),
        out_type=[
            jax.ShapeDtypeStruct((NW, NBINS), jnp.int32),
            jax.ShapeDtypeStruct((NW, L), jnp.int32),
        ],
        scratch_types=[
            pltpu.VMEM((NBINS,), jnp.int32),
            pltpu.VMEM((CHUNK,), jnp.int32),
            pltpu.VMEM((CHUNK,), jnp.int32),
            pltpu.VMEM((L,), jnp.int32),
            pltpu.SemaphoreType.DMA,
            pltpu.SemaphoreType.DMA,
        ],
    )
    def sc_hist(x_hbm, hist_hbm, max_hbm, hist_v, buf0, buf1, max_v, sem0, sem1):
        cid = lax.axis_index("c")
        sid = lax.axis_index("s")
        wid = sid * NC + cid
        base = wid * per_w

        zero16 = jnp.zeros((L,), jnp.int32)

        def zbody(i, carry):
            hist_v[pl.ds(i * L, L)] = zero16
            return carry
        lax.fori_loop(0, NBINS // L, zbody, 0)

        ones16 = jnp.ones((L,), jnp.int32)
        absmask = jnp.full((L,), 0x7FFFFFFF, jnp.int32)

        def process(buf, mx):
            def body(i, mx):
                bits = buf[pl.ds(i * L, L)] & absmask
                mx = jnp.maximum(mx, bits)
                idx = lax.shift_right_logical(bits, SHIFT)
                plsc.addupdate_scatter(hist_v, [idx], ones16)
                return mx
            return plsc.parallel_loop(0, CHUNK // L, unroll=8, carry=mx)(body)

        def start(c, buf, sem):
            pltpu.make_async_copy(
                x_hbm.at[pl.ds(base + c * CHUNK, CHUNK)], buf, sem).start()

        def wait(buf, sem):
            pltpu.make_async_copy(
                x_hbm.at[pl.ds(base, CHUNK)], buf, sem).wait()

        start(0, buf0, sem0)
        start(1, buf1, sem1)

        def pair(p, mx):
            wait(buf0, sem0)
            mx = process(buf0, mx)

            @pl.when(p + 1 < n_pairs)
            def _():
                start(2 * p + 2, buf0, sem0)

            wait(buf1, sem1)
            mx = process(buf1, mx)

            @pl.when(p + 1 < n_pairs)
            def _():
                start(2 * p + 3, buf1, sem1)

            return mx

        mx = lax.fori_loop(0, n_pairs, pair, jnp.zeros((L,), jnp.int32))

        max_v[...] = mx
        pltpu.sync_copy(max_v, max_hbm.at[wid])
        pltpu.sync_copy(hist_v, hist_hbm.at[wid])

    return sc_hist


def _make_tc_finalize(n):
    rows = NBINS // 128
    ks = [int(round(q * n)) - 1 for q in (0.9999, 0.999, 0.99)]
    half = 1 << (SHIFT - 1)

    def fin(hist_ref, max_ref, bufs_ref, out_ref):
        h = jnp.sum(hist_ref[...], axis=0)

        li = lax.broadcasted_iota(jnp.int32, (128, 128), 0)
        lj = lax.broadcasted_iota(jnp.int32, (128, 128), 1)
        lower = (li <= lj).astype(jnp.float32)

        def exact_dot(a_i32, m):
            acc = None
            for s in (16, 8, 0):
                d = ((lax.shift_right_logical(a_i32, s) & 255)
                     .astype(jnp.float32))
                p = lax.dot_general(d, m, (((1,), (0,)), ((), ())),
                                    preferred_element_type=jnp.float32)
                acc = p if acc is None else acc * 256.0 + p
            return acc

        rowcum = exact_dot(h, lower)
        rowsum = rowcum[:, 127:128].astype(jnp.int32)

        ri = lax.broadcasted_iota(jnp.int32, (rows, rows), 0)
        rj = lax.broadcasted_iota(jnp.int32, (rows, rows), 1)
        strict = (ri > rj).astype(jnp.float32)

        def exact_dot_l(m, a_i32):
            acc = None
            for s in (16, 8, 0):
                d = ((lax.shift_right_logical(a_i32, s) & 255)
                     .astype(jnp.float32))
                p = lax.dot_general(m, d, (((1,), (0,)), ((), ())),
                                    preferred_element_type=jnp.float32)
                acc = p if acc is None else acc * 256.0 + p
            return acc

        rowpre = exact_dot_l(strict, rowsum)
        cum = rowcum + rowpre

        def bin_of(k):
            return jnp.sum((cum <= float(k)).astype(jnp.int32))

        bins = [bin_of(k) for k in ks]

        maxbits = jnp.max(max_ref[...])

        lane = lax.broadcasted_iota(jnp.int32, (1, 128), 1)
        bits = jnp.where(lane == 0, maxbits, 0)
        for j, b in enumerate(bins):
            bits = jnp.where(lane == j + 1, b * (1 << SHIFT) + half, bits)
        vals = lax.bitcast_convert_type(bits, jnp.float32)

        bufv = jnp.zeros((1, 128), jnp.float32)
        for j in range(4):
            bufv = jnp.where(lane == j, bufs_ref[j], bufv)

        out_ref[...] = bufv * GAMMA + vals * (1.0 - GAMMA)

    return pl.pallas_call(
        fin,
        out_shape=jax.ShapeDtypeStruct((1, 128), jnp.float32),
        in_specs=[
            pl.BlockSpec(memory_space=pltpu.VMEM),
            pl.BlockSpec(memory_space=pltpu.VMEM),
            pl.BlockSpec(memory_space=pltpu.SMEM),
        ],
        out_specs=pl.BlockSpec(memory_space=pltpu.VMEM),
    )


def _make_tc_copy(shape):
    b0, b1, b2 = 1, shape[1] // 8, shape[2]

    def cp(x_ref, o_ref):
        o_ref[...] = x_ref[...]

    return pl.pallas_call(
        cp,
        grid=(shape[0], 8),
        in_specs=[pl.BlockSpec((b0, b1, b2), lambda i, j: (i, j, 0))],
        out_specs=pl.BlockSpec((b0, b1, b2), lambda i, j: (i, j, 0)),
        out_shape=jax.ShapeDtypeStruct(shape, jnp.float32),
    )


def kernel(x, max_buf, p99_99_buf, p99_9_buf, p99_buf):
    n = x.size
    xf = lax.bitcast_convert_type(x, jnp.int32).reshape(-1)
    hists, maxes = _make_sc_hist(n)(xf)
    bufs = jnp.stack([max_buf, p99_99_buf, p99_9_buf, p99_buf]).astype(jnp.float32)
    out = _make_tc_finalize(n)(hists.reshape(NW, NBINS // 128, 128), maxes, bufs)
    x_out = _make_tc_copy(x.shape)(x)
    return (x_out, out[0, 0], out[0, 1], out[0, 2], out[0, 3])


# speedup vs baseline: 201.9947x; 1.6116x over previous
import functools

import jax
import jax.numpy as jnp
from jax import lax
from jax.experimental import pallas as pl
from jax.experimental.pallas import tpu as pltpu
from jax.experimental.pallas import tpu_sc as plsc

GAMMA = 0.99

NC = 2
NS = 16
NW = NC * NS
L = 16

SHIFT = 15
NBINS = 1 << (31 - SHIFT)
HROWS = NBINS // 128
MINOR = 2048
CROWS = 8


def _make_sc_hist(n):
    rows = n // MINOR
    rows_per_w = rows // NW
    n_chunks = rows_per_w // CROWS
    n_pairs = n_chunks // 2
    assert rows * MINOR == n and n_chunks * CROWS == rows_per_w
    assert n_pairs * 2 == n_chunks

    mesh = plsc.VectorSubcoreMesh(core_axis_name="c", subcore_axis_name="s")

    @functools.partial(
        pl.kernel,
        mesh=mesh,
        compiler_params=pltpu.CompilerParams(needs_layout_passes=False),
        out_type=[
            jax.ShapeDtypeStruct((NW * HROWS, 128), jnp.int32),
            jax.ShapeDtypeStruct((NW, L), jnp.int32),
        ],
        scratch_types=[
            pltpu.VMEM((HROWS, 128), jnp.int32),
            pltpu.VMEM((CROWS, MINOR), jnp.int32),
            pltpu.VMEM((CROWS, MINOR), jnp.int32),
            pltpu.VMEM((L,), jnp.int32),
            pltpu.SemaphoreType.DMA,
            pltpu.SemaphoreType.DMA,
        ],
    )
    def sc_hist(x_hbm, hist_hbm, max_hbm, hist_v, buf0, buf1, max_v, sem0, sem1):
        cid = lax.axis_index("c")
        sid = lax.axis_index("s")
        wid = sid * NC + cid
        row_base = wid * rows_per_w

        zero16 = jnp.zeros((L,), jnp.int32)

        def zbody(i, carry):
            del carry
            hist_v[i // 8, pl.ds((i % 8) * L, L)] = zero16
            return 0
        lax.fori_loop(0, HROWS * 8, zbody, 0)

        ones16 = jnp.ones((L,), jnp.int32)
        absmask = jnp.full((L,), 0x7FFFFFFF, jnp.int32)

        def process(buf, mx):
            for r in range(CROWS):
                def body(i, mx, r=r):
                    bits = buf[r, pl.ds(i * L, L)] & absmask
                    mx = jnp.maximum(mx, bits)
                    idx = lax.shift_right_logical(bits, SHIFT)
                    plsc.addupdate_scatter(
                        hist_v,
                        [lax.shift_right_logical(idx, 7), idx & 127],
                        ones16)
                    return mx
                mx = plsc.parallel_loop(
                    0, MINOR // L, unroll=8, carry=mx)(body)
            return mx

        def start(c, buf, sem):
            pltpu.make_async_copy(
                x_hbm.at[pl.ds(row_base + c * CROWS, CROWS), :], buf,
                sem).start()

        def wait(buf, sem):
            pltpu.make_async_copy(
                x_hbm.at[pl.ds(row_base, CROWS), :], buf, sem).wait()

        start(0, buf0, sem0)
        start(1, buf1, sem1)

        def pair(p, mx):
            wait(buf0, sem0)
            mx = process(buf0, mx)

            @pl.when(p + 1 < n_pairs)
            def _():
                start(2 * p + 2, buf0, sem0)

            wait(buf1, sem1)
            mx = process(buf1, mx)

            @pl.when(p + 1 < n_pairs)
            def _():
                start(2 * p + 3, buf1, sem1)

            return mx

        mx = lax.fori_loop(0, n_pairs, pair, jnp.zeros((L,), jnp.int32))

        max_v[...] = mx
        pltpu.sync_copy(max_v, max_hbm.at[wid])
        pltpu.sync_copy(hist_v, hist_hbm.at[pl.ds(wid * HROWS, HROWS), :])

    return sc_hist


def _make_tc_finalize(n):
    rows = HROWS
    ks = [int(round(q * n)) - 1 for q in (0.9999, 0.999, 0.99)]
    half = 1 << (SHIFT - 1)

    def fin(hist_ref, max_ref, bufs_ref, out_ref):
        h = jnp.sum(hist_ref[...], axis=0)

        li = lax.broadcasted_iota(jnp.int32, (128, 128), 0)
        lj = lax.broadcasted_iota(jnp.int32, (128, 128), 1)
        lower = (li <= lj).astype(jnp.float32)

        def exact_dot(a_i32, m):
            acc = None
            for s in (16, 8, 0):
                d = ((lax.shift_right_logical(a_i32, s) & 255)
                     .astype(jnp.float32))
                p = lax.dot_general(d, m, (((1,), (0,)), ((), ())),
                                    preferred_element_type=jnp.float32)
                acc = p if acc is None else acc * 256.0 + p
            return acc

        rowcum = exact_dot(h, lower)
        rowsum = rowcum[:, 127:128].astype(jnp.int32)

        ri = lax.broadcasted_iota(jnp.int32, (rows, rows), 0)
        rj = lax.broadcasted_iota(jnp.int32, (rows, rows), 1)
        strict = (ri > rj).astype(jnp.float32)

        def exact_dot_l(m, a_i32):
            acc = None
            for s in (16, 8, 0):
                d = ((lax.shift_right_logical(a_i32, s) & 255)
                     .astype(jnp.float32))
                p = lax.dot_general(m, d, (((1,), (0,)), ((), ())),
                                    preferred_element_type=jnp.float32)
                acc = p if acc is None else acc * 256.0 + p
            return acc

        rowpre = exact_dot_l(strict, rowsum)
        cum = rowcum + rowpre

        def bin_of(k):
            return jnp.sum((cum <= float(k)).astype(jnp.int32))

        bins = [bin_of(k) for k in ks]

        maxbits = jnp.max(max_ref[...])

        lane = lax.broadcasted_iota(jnp.int32, (1, 128), 1)
        bits = jnp.where(lane == 0, maxbits, 0)
        for j, b in enumerate(bins):
            bits = jnp.where(lane == j + 1, b * (1 << SHIFT) + half, bits)
        vals = lax.bitcast_convert_type(bits, jnp.float32)

        bufv = jnp.zeros((1, 128), jnp.float32)
        for j in range(4):
            bufv = jnp.where(lane == j, bufs_ref[j], bufv)

        out_ref[...] = bufv * GAMMA + vals * (1.0 - GAMMA)

    return pl.pallas_call(
        fin,
        out_shape=jax.ShapeDtypeStruct((1, 128), jnp.float32),
        in_specs=[
            pl.BlockSpec(memory_space=pltpu.VMEM),
            pl.BlockSpec(memory_space=pltpu.VMEM),
            pl.BlockSpec(memory_space=pltpu.SMEM),
        ],
        out_specs=pl.BlockSpec(memory_space=pltpu.VMEM),
    )


def _make_tc_copy(shape):
    b0, b1, b2 = 1, shape[1] // 8, shape[2]

    def cp(x_ref, o_ref):
        o_ref[...] = x_ref[...]

    return pl.pallas_call(
        cp,
        grid=(shape[0], 8),
        in_specs=[pl.BlockSpec((b0, b1, b2), lambda i, j: (i, j, 0))],
        out_specs=pl.BlockSpec((b0, b1, b2), lambda i, j: (i, j, 0)),
        out_shape=jax.ShapeDtypeStruct(shape, jnp.float32),
    )


def kernel(x, max_buf, p99_99_buf, p99_9_buf, p99_buf):
    n = x.size
    x2d = lax.bitcast_convert_type(x, jnp.int32).reshape(n // MINOR, MINOR)
    hists, maxes = _make_sc_hist(n)(x2d)
    bufs = jnp.stack([max_buf, p99_99_buf, p99_9_buf, p99_buf]).astype(jnp.float32)
    out = _make_tc_finalize(n)(hists.reshape(NW, HROWS, 128), maxes, bufs)
    x_out = _make_tc_copy(x.shape)(x)
    return (x_out, out[0, 0], out[0, 1], out[0, 2], out[0, 3])


# speedup vs baseline: 292.3494x; 1.4473x over previous
import functools

import jax
import jax.numpy as jnp
from jax import lax
from jax.experimental import pallas as pl
from jax.experimental.pallas import tpu as pltpu
from jax.experimental.pallas import tpu_sc as plsc

GAMMA = 0.99

NC = 2
NS = 16
NW = NC * NS
L = 16

SHIFT = 15
NBINS = 1 << (31 - SHIFT)
HROWS = NBINS // 128
MINOR = 2048
CROWS = 8


def _make_sc_hist(shape):
    d0, d1, d2 = shape
    assert d2 == MINOR
    w_per_d0 = NW // d0
    rows_per_w = d1 // w_per_d0
    n_chunks = rows_per_w // CROWS
    n_pairs = n_chunks // 2
    assert n_chunks * CROWS == rows_per_w and n_pairs * 2 == n_chunks

    mesh = plsc.VectorSubcoreMesh(core_axis_name="c", subcore_axis_name="s")

    @functools.partial(
        pl.kernel,
        mesh=mesh,
        compiler_params=pltpu.CompilerParams(needs_layout_passes=False),
        out_type=[
            jax.ShapeDtypeStruct((NW * HROWS, 128), jnp.int32),
            jax.ShapeDtypeStruct((NW, L), jnp.int32),
        ],
        scratch_types=[
            pltpu.VMEM((HROWS, 128), jnp.int32),
            pltpu.VMEM((CROWS, MINOR), jnp.float32),
            pltpu.VMEM((CROWS, MINOR), jnp.float32),
            pltpu.VMEM((L,), jnp.int32),
            pltpu.SemaphoreType.DMA,
            pltpu.SemaphoreType.DMA,
        ],
    )
    def sc_hist(x_hbm, hist_hbm, max_hbm, hist_v, buf0, buf1, max_v, sem0, sem1):
        cid = lax.axis_index("c")
        sid = lax.axis_index("s")
        wid = sid * NC + cid
        maj = wid // w_per_d0
        row_base = (wid % w_per_d0) * rows_per_w

        zero16 = jnp.zeros((L,), jnp.int32)

        def zbody(i):
            hist_v[i // 8, pl.ds((i % 8) * L, L)] = zero16
        plsc.parallel_loop(0, HROWS * 8, unroll=8)(zbody)

        ones16 = jnp.ones((L,), jnp.int32)
        absmask = jnp.full((L,), 0x7FFFFFFF, jnp.int32)

        def process(buf, mx):
            for r in range(CROWS):
                def body(i, mx, r=r):
                    v = buf[r, pl.ds(i * L, L)]
                    bits = plsc.bitcast(v, jnp.int32) & absmask
                    mx = jnp.maximum(mx, bits)
                    idx = lax.shift_right_logical(bits, SHIFT)
                    plsc.addupdate_scatter(
                        hist_v,
                        [lax.shift_right_logical(idx, 7), idx & 127],
                        ones16)
                    return mx
                mx = plsc.parallel_loop(
                    0, MINOR // L, unroll=8, carry=mx)(body)
            return mx

        def start(c, buf, sem):
            pltpu.make_async_copy(
                x_hbm.at[maj, pl.ds(row_base + c * CROWS, CROWS), :], buf,
                sem).start()

        def wait(buf, sem):
            pltpu.make_async_copy(
                x_hbm.at[maj, pl.ds(row_base, CROWS), :], buf, sem).wait()

        start(0, buf0, sem0)
        start(1, buf1, sem1)

        def pair(p, mx):
            wait(buf0, sem0)
            mx = process(buf0, mx)

            @pl.when(p + 1 < n_pairs)
            def _():
                start(2 * p + 2, buf0, sem0)

            wait(buf1, sem1)
            mx = process(buf1, mx)

            @pl.when(p + 1 < n_pairs)
            def _():
                start(2 * p + 3, buf1, sem1)

            return mx

        mx = lax.fori_loop(0, n_pairs, pair, jnp.zeros((L,), jnp.int32))

        max_v[...] = mx
        pltpu.sync_copy(max_v, max_hbm.at[wid])
        pltpu.sync_copy(hist_v, hist_hbm.at[pl.ds(wid * HROWS, HROWS), :])

    return sc_hist


def _make_tc_finalize(n):
    rows = HROWS
    ks = [int(round(q * n)) - 1 for q in (0.9999, 0.999, 0.99)]
    half = 1 << (SHIFT - 1)

    def fin(hist_ref, max_ref, bufs_ref, out_ref):
        h = jnp.sum(hist_ref[...], axis=0)

        li = lax.broadcasted_iota(jnp.int32, (128, 128), 0)
        lj = lax.broadcasted_iota(jnp.int32, (128, 128), 1)
        lower = (li <= lj).astype(jnp.float32)

        def exact_dot(a_i32, m):
            acc = None
            for s in (16, 8, 0):
                d = ((lax.shift_right_logical(a_i32, s) & 255)
                     .astype(jnp.float32))
                p = lax.dot_general(d, m, (((1,), (0,)), ((), ())),
                                    preferred_element_type=jnp.float32)
                acc = p if acc is None else acc * 256.0 + p
            return acc

        rowcum = exact_dot(h, lower)
        rowsum = rowcum[:, 127:128].astype(jnp.int32)

        ri = lax.broadcasted_iota(jnp.int32, (rows, rows), 0)
        rj = lax.broadcasted_iota(jnp.int32, (rows, rows), 1)
        strict = (ri > rj).astype(jnp.float32)

        def exact_dot_l(m, a_i32):
            acc = None
            for s in (16, 8, 0):
                d = ((lax.shift_right_logical(a_i32, s) & 255)
                     .astype(jnp.float32))
                p = lax.dot_general(m, d, (((1,), (0,)), ((), ())),
                                    preferred_element_type=jnp.float32)
                acc = p if acc is None else acc * 256.0 + p
            return acc

        rowpre = exact_dot_l(strict, rowsum)
        cum = rowcum + rowpre

        def bin_of(k):
            return jnp.sum((cum <= float(k)).astype(jnp.int32))

        bins = [bin_of(k) for k in ks]

        maxbits = jnp.max(max_ref[...])

        lane = lax.broadcasted_iota(jnp.int32, (1, 128), 1)
        bits = jnp.where(lane == 0, maxbits, 0)
        for j, b in enumerate(bins):
            bits = jnp.where(lane == j + 1, b * (1 << SHIFT) + half, bits)
        vals = lax.bitcast_convert_type(bits, jnp.float32)

        bufv = jnp.zeros((1, 128), jnp.float32)
        for j in range(4):
            bufv = jnp.where(lane == j, bufs_ref[j], bufv)

        out_ref[...] = bufv * GAMMA + vals * (1.0 - GAMMA)

    return pl.pallas_call(
        fin,
        out_shape=jax.ShapeDtypeStruct((1, 128), jnp.float32),
        in_specs=[
            pl.BlockSpec(memory_space=pltpu.VMEM),
            pl.BlockSpec(memory_space=pltpu.VMEM),
            pl.BlockSpec(memory_space=pltpu.SMEM),
        ],
        out_specs=pl.BlockSpec(memory_space=pltpu.VMEM),
    )


def _make_tc_copy(shape):
    b0, b1, b2 = 1, shape[1] // 8, shape[2]

    def cp(x_ref, o_ref):
        o_ref[...] = x_ref[...]

    return pl.pallas_call(
        cp,
        grid=(shape[0], 8),
        in_specs=[pl.BlockSpec((b0, b1, b2), lambda i, j: (i, j, 0))],
        out_specs=pl.BlockSpec((b0, b1, b2), lambda i, j: (i, j, 0)),
        out_shape=jax.ShapeDtypeStruct(shape, jnp.float32),
    )


def kernel(x, max_buf, p99_99_buf, p99_9_buf, p99_buf):
    n = x.size
    hists, maxes = _make_sc_hist(x.shape)(x)
    bufs = jnp.stack([max_buf, p99_99_buf, p99_9_buf, p99_buf]).astype(jnp.float32)
    out = _make_tc_finalize(n)(hists.reshape(NW, HROWS, 128), maxes, bufs)
    x_out = _make_tc_copy(x.shape)(x)
    return (x_out, out[0, 0], out[0, 1], out[0, 2], out[0, 3])


# speedup vs baseline: 297.8497x; 1.0188x over previous
import functools

import jax
import jax.numpy as jnp
from jax import lax
from jax.experimental import pallas as pl
from jax.experimental.pallas import tpu as pltpu
from jax.experimental.pallas import tpu_sc as plsc

GAMMA = 0.99

NC = 2
NS = 16
NW = NC * NS
L = 16

SHIFT = 15
NBINS = 1 << (31 - SHIFT)
HROWS = NBINS // 128
MINOR = 2048
CROWS = 8


def _make_sc_hist(shape):
    d0, d1, d2 = shape
    assert d2 == MINOR
    w_per_d0 = NW // d0
    rows_per_w = d1 // w_per_d0
    n_chunks = rows_per_w // CROWS
    n_pairs = n_chunks // 2
    assert n_chunks * CROWS == rows_per_w and n_pairs * 2 == n_chunks

    mesh = plsc.VectorSubcoreMesh(core_axis_name="c", subcore_axis_name="s")

    @functools.partial(
        pl.kernel,
        mesh=mesh,
        compiler_params=pltpu.CompilerParams(needs_layout_passes=False),
        out_type=jax.ShapeDtypeStruct((NW * HROWS, 128), jnp.int32),
        scratch_types=[
            pltpu.VMEM((HROWS, 128), jnp.int32),
            pltpu.VMEM((CROWS, MINOR), jnp.float32),
            pltpu.VMEM((CROWS, MINOR), jnp.float32),
            pltpu.SemaphoreType.DMA,
            pltpu.SemaphoreType.DMA,
        ],
    )
    def sc_hist(x_hbm, hist_hbm, hist_v, buf0, buf1, sem0, sem1):
        cid = lax.axis_index("c")
        sid = lax.axis_index("s")
        wid = sid * NC + cid
        maj = wid // w_per_d0
        row_base = (wid % w_per_d0) * rows_per_w

        def start(c, buf, sem):
            pltpu.make_async_copy(
                x_hbm.at[maj, pl.ds(row_base + c * CROWS, CROWS), :], buf,
                sem).start()

        def wait(buf, sem):
            pltpu.make_async_copy(
                x_hbm.at[maj, pl.ds(row_base, CROWS), :], buf, sem).wait()

        start(0, buf0, sem0)
        start(1, buf1, sem1)

        zero16 = jnp.zeros((L,), jnp.int32)

        def zbody(i):
            hist_v[i // 8, pl.ds((i % 8) * L, L)] = zero16
        plsc.parallel_loop(0, HROWS * 8, unroll=8)(zbody)

        ones16 = jnp.ones((L,), jnp.int32)
        absmask = jnp.full((L,), 0x7FFFFFFF, jnp.int32)

        def process(buf):
            for r in range(CROWS):
                def body(i, r=r):
                    v = buf[r, pl.ds(i * L, L)]
                    bits = plsc.bitcast(v, jnp.int32) & absmask
                    idx = lax.shift_right_logical(bits, SHIFT)
                    plsc.addupdate_scatter(
                        hist_v,
                        [lax.shift_right_logical(idx, 7), idx & 127],
                        ones16)
                plsc.parallel_loop(0, MINOR // L, unroll=16)(body)

        def pair(p, carry):
            wait(buf0, sem0)
            process(buf0)

            @pl.when(p + 1 < n_pairs)
            def _():
                start(2 * p + 2, buf0, sem0)

            wait(buf1, sem1)
            process(buf1)

            @pl.when(p + 1 < n_pairs)
            def _():
                start(2 * p + 3, buf1, sem1)

            return carry

        lax.fori_loop(0, n_pairs, pair, 0)

        pltpu.sync_copy(hist_v, hist_hbm.at[pl.ds(wid * HROWS, HROWS), :])

    return sc_hist


def _make_tc_finalize(n):
    rows = HROWS
    ks = [n - 1] + [int(round(q * n)) - 1 for q in (0.9999, 0.999, 0.99)]
    half = 1 << (SHIFT - 1)

    def fin(hist_ref, bufs_ref, out_ref):
        h = jnp.sum(hist_ref[...], axis=0)

        li = lax.broadcasted_iota(jnp.int32, (128, 128), 0)
        lj = lax.broadcasted_iota(jnp.int32, (128, 128), 1)
        lower = (li <= lj).astype(jnp.float32)

        def exact_dot(a_i32, m):
            acc = None
            for s in (16, 8, 0):
                d = ((lax.shift_right_logical(a_i32, s) & 255)
                     .astype(jnp.float32))
                p = lax.dot_general(d, m, (((1,), (0,)), ((), ())),
                                    preferred_element_type=jnp.float32)
                acc = p if acc is None else acc * 256.0 + p
            return acc

        rowcum = exact_dot(h, lower)
        rowsum = rowcum[:, 127:128].astype(jnp.int32)

        ri = lax.broadcasted_iota(jnp.int32, (rows, rows), 0)
        rj = lax.broadcasted_iota(jnp.int32, (rows, rows), 1)
        strict = (ri > rj).astype(jnp.float32)

        def exact_dot_l(m, a_i32):
            acc = None
            for s in (16, 8, 0):
                d = ((lax.shift_right_logical(a_i32, s) & 255)
                     .astype(jnp.float32))
                p = lax.dot_general(m, d, (((1,), (0,)), ((), ())),
                                    preferred_element_type=jnp.float32)
                acc = p if acc is None else acc * 256.0 + p
            return acc

        rowpre = exact_dot_l(strict, rowsum)
        cum = rowcum + rowpre

        def bin_of(k):
            return jnp.sum((cum <= float(k)).astype(jnp.int32))

        bins = [bin_of(k) for k in ks]

        lane = lax.broadcasted_iota(jnp.int32, (1, 128), 1)
        bits = jnp.zeros((1, 128), jnp.int32)
        for j, b in enumerate(bins):
            bits = jnp.where(lane == j, b * (1 << SHIFT) + half, bits)
        vals = lax.bitcast_convert_type(bits, jnp.float32)

        bufv = jnp.zeros((1, 128), jnp.float32)
        for j in range(4):
            bufv = jnp.where(lane == j, bufs_ref[j], bufv)

        out_ref[...] = bufv * GAMMA + vals * (1.0 - GAMMA)

    return pl.pallas_call(
        fin,
        out_shape=jax.ShapeDtypeStruct((1, 128), jnp.float32),
        in_specs=[
            pl.BlockSpec(memory_space=pltpu.VMEM),
            pl.BlockSpec(memory_space=pltpu.SMEM),
        ],
        out_specs=pl.BlockSpec(memory_space=pltpu.VMEM),
    )


def _make_tc_copy(shape):
    b0, b1, b2 = 1, shape[1] // 8, shape[2]

    def cp(x_ref, o_ref):
        o_ref[...] = x_ref[...]

    return pl.pallas_call(
        cp,
        grid=(shape[0], 8),
        in_specs=[pl.BlockSpec((b0, b1, b2), lambda i, j: (i, j, 0))],
        out_specs=pl.BlockSpec((b0, b1, b2), lambda i, j: (i, j, 0)),
        out_shape=jax.ShapeDtypeStruct(shape, jnp.float32),
    )


def kernel(x, max_buf, p99_99_buf, p99_9_buf, p99_buf):
    n = x.size
    hists = _make_sc_hist(x.shape)(x)
    bufs = jnp.stack([max_buf, p99_99_buf, p99_9_buf, p99_buf]).astype(jnp.float32)
    out = _make_tc_finalize(n)(hists.reshape(NW, HROWS, 128), bufs)
    x_out = _make_tc_copy(x.shape)(x)
    return (x_out, out[0, 0], out[0, 1], out[0, 2], out[0, 3])


# speedup vs baseline: 300.8329x; 1.0100x over previous
import functools

import jax
import jax.numpy as jnp
from jax import lax
from jax.experimental import pallas as pl
from jax.experimental.pallas import tpu as pltpu
from jax.experimental.pallas import tpu_sc as plsc

GAMMA = 0.99

NC = 2
NS = 16
NW = NC * NS
L = 16

SHIFT = 16
NBINS = 1 << (31 - SHIFT)
HROWS = NBINS // 128
MINOR = 2048
CROWS = 8
NBUF = 4


def _make_sc_hist(shape):
    d0, d1, d2 = shape
    assert d2 == MINOR
    w_per_d0 = NW // d0
    rows_per_w = d1 // w_per_d0
    n_chunks = rows_per_w // CROWS
    n_groups = n_chunks // NBUF
    assert n_chunks * CROWS == rows_per_w and n_groups * NBUF == n_chunks

    mesh = plsc.VectorSubcoreMesh(core_axis_name="c", subcore_axis_name="s")

    @functools.partial(
        pl.kernel,
        mesh=mesh,
        compiler_params=pltpu.CompilerParams(needs_layout_passes=False),
        out_type=jax.ShapeDtypeStruct((NW * HROWS, 128), jnp.int32),
        scratch_types=(
            [pltpu.VMEM((HROWS, 128), jnp.int32)]
            + [pltpu.VMEM((CROWS, MINOR), jnp.float32) for _ in range(NBUF)]
            + [pltpu.SemaphoreType.DMA for _ in range(NBUF)]
        ),
    )
    def sc_hist(x_hbm, hist_hbm, hist_v, *bufs_and_sems):
        bufs = bufs_and_sems[:NBUF]
        sems = bufs_and_sems[NBUF:]
        cid = lax.axis_index("c")
        sid = lax.axis_index("s")
        wid = sid * NC + cid
        maj = wid // w_per_d0
        row_base = (wid % w_per_d0) * rows_per_w

        def start(c, buf, sem):
            pltpu.make_async_copy(
                x_hbm.at[maj, pl.ds(row_base + c * CROWS, CROWS), :], buf,
                sem).start()

        def wait(buf, sem):
            pltpu.make_async_copy(
                x_hbm.at[maj, pl.ds(row_base, CROWS), :], buf, sem).wait()

        for b in range(NBUF):
            start(b, bufs[b], sems[b])

        zero16 = jnp.zeros((L,), jnp.int32)

        def zbody(i):
            hist_v[i // 8, pl.ds((i % 8) * L, L)] = zero16
        plsc.parallel_loop(0, HROWS * 8, unroll=8)(zbody)

        ones16 = jnp.ones((L,), jnp.int32)
        absmask = jnp.full((L,), 0x7FFFFFFF, jnp.int32)

        def process(buf):
            for r in range(CROWS):
                def body(i, r=r):
                    v = buf[r, pl.ds(i * L, L)]
                    bits = plsc.bitcast(v, jnp.int32) & absmask
                    idx = lax.shift_right_logical(bits, SHIFT)
                    plsc.addupdate_scatter(
                        hist_v,
                        [lax.shift_right_logical(idx, 7), idx & 127],
                        ones16)
                plsc.parallel_loop(0, MINOR // L, unroll=16)(body)

        def group(g, carry):
            for b in range(NBUF):
                wait(bufs[b], sems[b])
                process(bufs[b])

                @pl.when(g * NBUF + b + NBUF < n_chunks)
                def _(b=b):
                    start(g * NBUF + b + NBUF, bufs[b], sems[b])

            return carry

        lax.fori_loop(0, n_groups, group, 0)

        pltpu.sync_copy(hist_v, hist_hbm.at[pl.ds(wid * HROWS, HROWS), :])

    return sc_hist


def _make_tc_finalize(n):
    rows = HROWS
    ks = [n - 1] + [int(round(q * n)) - 1 for q in (0.9999, 0.999, 0.99)]
    half = 1 << (SHIFT - 1)

    def fin(hist_ref, bufs_ref, out_ref):
        h = jnp.sum(hist_ref[...], axis=0)

        li = lax.broadcasted_iota(jnp.int32, (128, 128), 0)
        lj = lax.broadcasted_iota(jnp.int32, (128, 128), 1)
        lower = (li <= lj).astype(jnp.float32)

        def exact_dot(a_i32, m):
            acc = None
            for s in (16, 8, 0):
                d = ((lax.shift_right_logical(a_i32, s) & 255)
                     .astype(jnp.float32))
                p = lax.dot_general(d, m, (((1,), (0,)), ((), ())),
                                    preferred_element_type=jnp.float32)
                acc = p if acc is None else acc * 256.0 + p
            return acc

        rowcum = exact_dot(h, lower)
        rowsum = rowcum[:, 127:128].astype(jnp.int32)

        ri = lax.broadcasted_iota(jnp.int32, (rows, rows), 0)
        rj = lax.broadcasted_iota(jnp.int32, (rows, rows), 1)
        strict = (ri > rj).astype(jnp.float32)

        def exact_dot_l(m, a_i32):
            acc = None
            for s in (16, 8, 0):
                d = ((lax.shift_right_logical(a_i32, s) & 255)
                     .astype(jnp.float32))
                p = lax.dot_general(m, d, (((1,), (0,)), ((), ())),
                                    preferred_element_type=jnp.float32)
                acc = p if acc is None else acc * 256.0 + p
            return acc

        rowpre = exact_dot_l(strict, rowsum)
        cum = rowcum + rowpre

        def bin_of(k):
            return jnp.sum((cum <= float(k)).astype(jnp.int32))

        bins = [bin_of(k) for k in ks]

        lane = lax.broadcasted_iota(jnp.int32, (1, 128), 1)
        bits = jnp.zeros((1, 128), jnp.int32)
        for j, b in enumerate(bins):
            bits = jnp.where(lane == j, b * (1 << SHIFT) + half, bits)
        vals = lax.bitcast_convert_type(bits, jnp.float32)

        bufv = jnp.zeros((1, 128), jnp.float32)
        for j in range(4):
            bufv = jnp.where(lane == j, bufs_ref[j], bufv)

        out_ref[...] = bufv * GAMMA + vals * (1.0 - GAMMA)

    return pl.pallas_call(
        fin,
        out_shape=jax.ShapeDtypeStruct((1, 128), jnp.float32),
        in_specs=[
            pl.BlockSpec(memory_space=pltpu.VMEM),
            pl.BlockSpec(memory_space=pltpu.SMEM),
        ],
        out_specs=pl.BlockSpec(memory_space=pltpu.VMEM),
    )


def _make_tc_copy(shape):
    b0, b1, b2 = 1, shape[1] // 8, shape[2]

    def cp(x_ref, o_ref):
        o_ref[...] = x_ref[...]

    return pl.pallas_call(
        cp,
        grid=(shape[0], 8),
        in_specs=[pl.BlockSpec((b0, b1, b2), lambda i, j: (i, j, 0))],
        out_specs=pl.BlockSpec((b0, b1, b2), lambda i, j: (i, j, 0)),
        out_shape=jax.ShapeDtypeStruct(shape, jnp.float32),
    )


def kernel(x, max_buf, p99_99_buf, p99_9_buf, p99_buf):
    n = x.size
    hists = _make_sc_hist(x.shape)(x)
    bufs = jnp.stack([max_buf, p99_99_buf, p99_9_buf, p99_buf]).astype(jnp.float32)
    out = _make_tc_finalize(n)(hists.reshape(NW, HROWS, 128), bufs)
    x_out = _make_tc_copy(x.shape)(x)
    return (x_out, out[0, 0], out[0, 1], out[0, 2], out[0, 3])
